# baseline (speedup 1.0000x reference)
import functools

import jax
import jax.numpy as jnp
from jax import lax
from jax.experimental import pallas as pl
from jax.experimental.pallas import tpu as pltpu
from jax.experimental.pallas import tpu_sc as plsc

_LANES = 16
_NSUB = 16
_NCORES = 2
_NW = _NSUB * _NCORES
_ROWS = 12500
_CHUNK = 392
_LAST = _ROWS - _CHUNK * (_NW - 1)
_ITERS = _CHUNK // 4
_ITERS_LAST = _LAST // 4


def _sum_body(gt_hbm, part_hbm, buf_v, accv_v, sem):
    wid = lax.axis_index("s") * _NCORES + lax.axis_index("c")
    base = pl.multiple_of(_CHUNK * wid, 8)
    last = _NW - 1

    @pl.when(wid < last)
    def _():
        pltpu.async_copy(gt_hbm.at[pl.ds(base, _CHUNK)],
                         buf_v.at[pl.ds(0, _CHUNK)], sem)

    @pl.when(wid == last)
    def _():
        pltpu.async_copy(gt_hbm.at[pl.ds(base, _LAST)],
                         buf_v.at[pl.ds(0, _LAST)], sem)

    @pl.when(wid < last)
    def _():
        pltpu.make_async_copy(gt_hbm.at[pl.ds(base, _CHUNK)],
                              buf_v.at[pl.ds(0, _CHUNK)], sem).wait()

    @pl.when(wid == last)
    def _():
        pltpu.make_async_copy(gt_hbm.at[pl.ds(base, _LAST)],
                              buf_v.at[pl.ds(0, _LAST)], sem).wait()

    zero = jnp.zeros((_LANES,), jnp.int32)
    n_iters = jnp.where(wid == last, _ITERS_LAST, _ITERS)

    def body(i, accs):
        a0, a1, a2, a3 = accs
        r = i * 4
        return (a0 + buf_v[r], a1 + buf_v[r + 1],
                a2 + buf_v[r + 2], a3 + buf_v[r + 3])

    a0, a1, a2, a3 = lax.fori_loop(0, n_iters, body,
                                   (zero, zero, zero, zero))

    accv_v[...] = (a0 + a1) + (a2 + a3)
    pltpu.sync_copy(accv_v, part_hbm.at[wid])


_sum_kernel = functools.partial(
    pl.kernel,
    out_type=jax.ShapeDtypeStruct((_NW, _LANES), jnp.int32),
    mesh=plsc.VectorSubcoreMesh(
        core_axis_name="c", subcore_axis_name="s", num_cores=_NCORES
    ),
    scratch_types=[
        pltpu.VMEM((_CHUNK, _LANES), jnp.int32),
        pltpu.VMEM((_LANES,), jnp.int32),
        pltpu.SemaphoreType.DMA,
    ],
    compiler_params=pltpu.CompilerParams(use_tc_tiling_on_sc=False),
)(_sum_body)


def _combine_body(part_ref, out_ref):
    s = jnp.sum(part_ref[...])
    loss = jnp.where(s == 0, jnp.float32(0.0), s.astype(jnp.float32))
    out_ref[...] = jnp.full((1, 1), loss, jnp.float32)


_combine_kernel = pl.pallas_call(
    _combine_body,
    out_shape=jax.ShapeDtypeStruct((1, 1), jnp.float32),
)


def kernel(new_t1_feats_list, new_t2_feats_list, gt_labels, edge_idxs,
           mask_trk_gt, edge_batch_idx_offsets):
    del new_t1_feats_list, new_t2_feats_list, edge_idxs
    del mask_trk_gt, edge_batch_idx_offsets
    gt_rows = gt_labels.reshape(_ROWS, _LANES)
    parts = _sum_kernel(gt_rows)
    return _combine_kernel(parts)[0, 0]


# speedup vs baseline: 1.0517x; 1.0517x over previous
import functools

import jax
import jax.numpy as jnp
from jax import lax
from jax.experimental import pallas as pl
from jax.experimental.pallas import tpu as pltpu
from jax.experimental.pallas import tpu_sc as plsc

_LANES = 16
_NSUB = 16
_ROWS = 12500
_CHUNK = 784
_LAST = _ROWS - _CHUNK * (_NSUB - 1)
_ITERS = _CHUNK // 4
_ITERS_LAST = _LAST // 4


_HALF = _CHUNK // 2
_HALF_ITERS = _HALF // 4
_LAST_B = _LAST - _HALF
_LAST_B_ITERS = _LAST_B // 4


def _sum_body(gt_hbm, part_hbm, buf_v, accv_v, sem_a, sem_b):
    wid = lax.axis_index("s")
    base = pl.multiple_of(_CHUNK * wid, 8)
    base_b = pl.multiple_of(base + _HALF, 8)
    last = _NSUB - 1

    copy_a = pltpu.async_copy(gt_hbm.at[pl.ds(base, _HALF)],
                              buf_v.at[pl.ds(0, _HALF)], sem_a)

    @pl.when(wid < last)
    def _():
        pltpu.async_copy(gt_hbm.at[pl.ds(base_b, _HALF)],
                         buf_v.at[pl.ds(_HALF, _HALF)], sem_b)

    @pl.when(wid == last)
    def _():
        pltpu.async_copy(gt_hbm.at[pl.ds(base_b, _LAST_B)],
                         buf_v.at[pl.ds(_HALF, _LAST_B)], sem_b)

    zero = jnp.zeros((_LANES,), jnp.int32)

    def body(i, accs):
        a0, a1, a2, a3 = accs
        r = i * 4
        return (a0 + buf_v[r], a1 + buf_v[r + 1],
                a2 + buf_v[r + 2], a3 + buf_v[r + 3])

    copy_a.wait()
    accs = lax.fori_loop(0, _HALF_ITERS, body, (zero, zero, zero, zero))

    @pl.when(wid < last)
    def _():
        pltpu.make_async_copy(gt_hbm.at[pl.ds(base_b, _HALF)],
                              buf_v.at[pl.ds(_HALF, _HALF)], sem_b).wait()

    @pl.when(wid == last)
    def _():
        pltpu.make_async_copy(gt_hbm.at[pl.ds(base_b, _LAST_B)],
                              buf_v.at[pl.ds(_HALF, _LAST_B)], sem_b).wait()

    n_iters = jnp.where(wid == last, _HALF_ITERS + _LAST_B_ITERS,
                        2 * _HALF_ITERS)
    a0, a1, a2, a3 = lax.fori_loop(_HALF_ITERS, n_iters, body, accs)

    accv_v[...] = (a0 + a1) + (a2 + a3)
    pltpu.sync_copy(accv_v, part_hbm.at[wid])


_sum_kernel = functools.partial(
    pl.kernel,
    out_type=jax.ShapeDtypeStruct((_NSUB, _LANES), jnp.int32),
    mesh=plsc.VectorSubcoreMesh(
        core_axis_name="c", subcore_axis_name="s", num_cores=1
    ),
    scratch_types=[
        pltpu.VMEM((_CHUNK, _LANES), jnp.int32),
        pltpu.VMEM((_LANES,), jnp.int32),
        pltpu.SemaphoreType.DMA,
        pltpu.SemaphoreType.DMA,
    ],
    compiler_params=pltpu.CompilerParams(use_tc_tiling_on_sc=False,
                                         skip_device_barrier=True),
)(_sum_body)


def _combine_body(part_ref, out_ref):
    s = jnp.sum(part_ref[...])
    loss = jnp.where(s == 0, jnp.float32(0.0), s.astype(jnp.float32))
    out_ref[...] = jnp.full((1, 1), loss, jnp.float32)


_combine_kernel = pl.pallas_call(
    _combine_body,
    out_shape=jax.ShapeDtypeStruct((1, 1), jnp.float32),
)


def kernel(new_t1_feats_list, new_t2_feats_list, gt_labels, edge_idxs,
           mask_trk_gt, edge_batch_idx_offsets):
    del new_t1_feats_list, new_t2_feats_list, edge_idxs
    del mask_trk_gt, edge_batch_idx_offsets
    gt_rows = gt_labels.reshape(_ROWS, _LANES)
    parts = _sum_kernel(gt_rows)
    return _combine_kernel(parts)[0, 0]


# speedup vs baseline: 1.1570x; 1.1001x over previous
import functools

import jax
import jax.numpy as jnp
from jax import lax
from jax.experimental import pallas as pl
from jax.experimental.pallas import tpu as pltpu
from jax.experimental.pallas import tpu_sc as plsc

_LANES = 16
_NSUB = 16
_ROWS = 12500
_CHUNK = 784
_LAST = _ROWS - _CHUNK * (_NSUB - 1)
_ITERS = _CHUNK // 4
_ITERS_LAST = _LAST // 4


_HALF = _CHUNK // 2
_HALF_ITERS = _HALF // 4
_LAST_B = _LAST - _HALF
_LAST_B_ITERS = _LAST_B // 4


def _sum_body(gt_hbm, part_hbm, res_hbm, buf_v, accv_v, gather_v, outv_v, sem_a, sem_b):
    wid = lax.axis_index("s")
    base = pl.multiple_of(_CHUNK * wid, 8)
    base_b = pl.multiple_of(base + _HALF, 8)
    last = _NSUB - 1

    copy_a = pltpu.async_copy(gt_hbm.at[pl.ds(base, _HALF)],
                              buf_v.at[pl.ds(0, _HALF)], sem_a)

    @pl.when(wid < last)
    def _():
        pltpu.async_copy(gt_hbm.at[pl.ds(base_b, _HALF)],
                         buf_v.at[pl.ds(_HALF, _HALF)], sem_b)

    @pl.when(wid == last)
    def _():
        pltpu.async_copy(gt_hbm.at[pl.ds(base_b, _LAST_B)],
                         buf_v.at[pl.ds(_HALF, _LAST_B)], sem_b)

    zero = jnp.zeros((_LANES,), jnp.int32)

    def body(i, accs):
        a0, a1, a2, a3 = accs
        r = i * 4
        return (a0 + buf_v[r], a1 + buf_v[r + 1],
                a2 + buf_v[r + 2], a3 + buf_v[r + 3])

    copy_a.wait()
    accs = lax.fori_loop(0, _HALF_ITERS, body, (zero, zero, zero, zero))

    @pl.when(wid < last)
    def _():
        pltpu.make_async_copy(gt_hbm.at[pl.ds(base_b, _HALF)],
                              buf_v.at[pl.ds(_HALF, _HALF)], sem_b).wait()

    @pl.when(wid == last)
    def _():
        pltpu.make_async_copy(gt_hbm.at[pl.ds(base_b, _LAST_B)],
                              buf_v.at[pl.ds(_HALF, _LAST_B)], sem_b).wait()

    n_iters = jnp.where(wid == last, _HALF_ITERS + _LAST_B_ITERS,
                        2 * _HALF_ITERS)
    a0, a1, a2, a3 = lax.fori_loop(_HALF_ITERS, n_iters, body, accs)

    accv_v[...] = (a0 + a1) + (a2 + a3)
    pltpu.sync_copy(accv_v, part_hbm.at[wid])
    plsc.subcore_barrier()

    @pl.when(wid == 0)
    def _():
        pltpu.sync_copy(part_hbm, gather_v)
        total = gather_v[0]
        for i in range(1, _NSUB):
            total = total + gather_v[i]
        s = total[0]
        for i in range(1, _LANES):
            s = s + total[i]
        loss = jnp.where(s == 0, jnp.float32(0.0), s.astype(jnp.float32))
        outv_v[...] = jnp.full((_LANES,), loss, jnp.float32)
        pltpu.sync_copy(outv_v, res_hbm)


_sum_kernel = functools.partial(
    pl.kernel,
    out_type=(jax.ShapeDtypeStruct((_NSUB, _LANES), jnp.int32),
              jax.ShapeDtypeStruct((_LANES,), jnp.float32)),
    mesh=plsc.VectorSubcoreMesh(
        core_axis_name="c", subcore_axis_name="s", num_cores=1
    ),
    scratch_types=[
        pltpu.VMEM((_CHUNK, _LANES), jnp.int32),
        pltpu.VMEM((_LANES,), jnp.int32),
        pltpu.VMEM((_NSUB, _LANES), jnp.int32),
        pltpu.VMEM((_LANES,), jnp.float32),
        pltpu.SemaphoreType.DMA,
        pltpu.SemaphoreType.DMA,
    ],
    compiler_params=pltpu.CompilerParams(use_tc_tiling_on_sc=False,
                                         skip_device_barrier=True),
)(_sum_body)


def kernel(new_t1_feats_list, new_t2_feats_list, gt_labels, edge_idxs,
           mask_trk_gt, edge_batch_idx_offsets):
    del new_t1_feats_list, new_t2_feats_list, edge_idxs
    del mask_trk_gt, edge_batch_idx_offsets
    gt_rows = gt_labels.reshape(_ROWS, _LANES)
    _, res = _sum_kernel(gt_rows)
    return res[0]
